# Initial kernel scaffold; baseline (speedup 1.0000x reference)
import jax
import jax.numpy as jnp
from jax.experimental import pallas as pl


def kernel(user_embed, item_embed, W, edge_index, edge_weight):
    raise NotImplementedError("write your pallas kernel here")


# speedup vs baseline: 4.0305x; 4.0305x over previous
import functools

import jax
import jax.numpy as jnp
from jax import lax
from jax.experimental import pallas as pl
from jax.experimental.pallas import tpu as pltpu
from jax.experimental.pallas import tpu_sc as plsc

NUM_USERS = 5000
NUM_ITEMS = 5000
N = NUM_USERS + NUM_ITEMS
E = 320000
D = 128
BD = 64
CL_EPS = 0.2

NC = 2
NS = 16
NW = NC * NS
CHUNK = 128
NCH = -(-E // (NW * CHUNK))
EPAD = NW * NCH * CHUNK
RPT = 632
NP = NS * RPT


def _make_spmm():
    mesh = plsc.VectorSubcoreMesh(core_axis_name="c", subcore_axis_name="s")

    @functools.partial(
        pl.kernel,
        out_type=jax.ShapeDtypeStruct((NC, NP, D), jnp.float32),
        mesh=mesh,
        scratch_types=[
            pltpu.VMEM((NCH, CHUNK), jnp.int32),
            pltpu.VMEM((NCH, CHUNK), jnp.int32),
            pltpu.VMEM((NCH, CHUNK), jnp.float32),
            pltpu.VMEM((CHUNK, D), jnp.float32),
            pltpu.VMEM_SHARED((NP, D), jnp.float32),
            pltpu.SemaphoreType.DMA,
        ],
    )
    def spmm(x_hbm, srcm, dstm, wm, y_hbm, src_v, dst_v, w_v, gbuf, acc, sem):
        cid = lax.axis_index("c")
        sid = lax.axis_index("s")
        wid = cid * NS + sid

        pltpu.sync_copy(srcm.at[wid], src_v)
        pltpu.sync_copy(dstm.at[wid], dst_v)
        pltpu.sync_copy(wm.at[wid], w_v)

        zero16 = jnp.zeros((16,), jnp.float32)

        def _zrow(r, carry):
            for g in range(8):
                gbuf[r, pl.ds(g * 16, 16)] = zero16
            return carry

        lax.fori_loop(0, CHUNK, _zrow, 0)
        base = sid * RPT
        for k in range(RPT // CHUNK):
            pltpu.sync_copy(gbuf, acc.at[pl.ds(base + k * CHUNK, CHUNK)])
        rem = RPT % CHUNK
        if rem:
            pltpu.sync_copy(gbuf.at[pl.ds(0, rem)],
                            acc.at[pl.ds(base + RPT - rem, rem)])
        plsc.subcore_barrier()

        def _chunk(j, carry):
            pltpu.async_copy(x_hbm.at[src_v.at[j]], gbuf, sem).wait()

            def _grp(g, c2):
                w16 = w_v[j, pl.ds(g * 16, 16)]
                for lane in range(16):
                    w = w16[lane]
                    e = g * 16 + lane
                    for q in range(8):
                        gbuf[e, pl.ds(q * 16, 16)] = (
                            gbuf[e, pl.ds(q * 16, 16)] * w)
                return c2

            lax.fori_loop(0, CHUNK // 16, _grp, 0)
            pltpu.sync_copy(gbuf, acc.at[dst_v.at[j]], add=True)
            return carry

        lax.fori_loop(0, NCH, _chunk, 0)
        plsc.subcore_barrier()

        pltpu.sync_copy(acc.at[pl.ds(base, RPT)],
                        y_hbm.at[cid, pl.ds(base, RPT)])

    return spmm


_spmm = _make_spmm()

_ROWS = 1000


def _bin_body(x_ref, wt_ref, c_ref, o_ref):
    s = jnp.sign(jnp.dot(x_ref[...], wt_ref[...],
                         preferred_element_type=jnp.float32))
    o_ref[...] = s * c_ref[...]


def _bin_call(x, wt, c):
    return pl.pallas_call(
        _bin_body,
        grid=(N // _ROWS,),
        in_specs=[
            pl.BlockSpec((_ROWS, D), lambda i: (i, 0)),
            pl.BlockSpec((D, BD), lambda i: (0, 0)),
            pl.BlockSpec((_ROWS, BD), lambda i: (i, 0)),
        ],
        out_specs=pl.BlockSpec((_ROWS, BD), lambda i: (i, 0)),
        out_shape=jax.ShapeDtypeStruct((N, BD), jnp.float32),
    )(x, wt, c)


def _merge_bin_body(p_ref, wt_ref, c_ref, x_ref, o_ref):
    x = p_ref[0] + p_ref[1]
    x_ref[...] = x
    s = jnp.sign(jnp.dot(x, wt_ref[...], preferred_element_type=jnp.float32))
    o_ref[...] = s * c_ref[...]


def _merge_bin_call(p, wt, c):
    return pl.pallas_call(
        _merge_bin_body,
        grid=(N // _ROWS,),
        in_specs=[
            pl.BlockSpec((NC, _ROWS, D), lambda i: (0, i, 0)),
            pl.BlockSpec((D, BD), lambda i: (0, 0)),
            pl.BlockSpec((_ROWS, BD), lambda i: (i, 0)),
        ],
        out_specs=[
            pl.BlockSpec((_ROWS, D), lambda i: (i, 0)),
            pl.BlockSpec((_ROWS, BD), lambda i: (i, 0)),
        ],
        out_shape=[
            jax.ShapeDtypeStruct((N, D), jnp.float32),
            jax.ShapeDtypeStruct((N, BD), jnp.float32),
        ],
    )(p, wt, c)


def _merge_bin_last_body(p_ref, wt_ref, c_ref, o_ref):
    x = p_ref[0] + p_ref[1]
    s = jnp.sign(jnp.dot(x, wt_ref[...], preferred_element_type=jnp.float32))
    o_ref[...] = s * c_ref[...]


def _merge_bin_last_call(p, wt, c):
    return pl.pallas_call(
        _merge_bin_last_body,
        grid=(N // _ROWS,),
        in_specs=[
            pl.BlockSpec((NC, _ROWS, D), lambda i: (0, i, 0)),
            pl.BlockSpec((D, BD), lambda i: (0, 0)),
            pl.BlockSpec((_ROWS, BD), lambda i: (i, 0)),
        ],
        out_specs=pl.BlockSpec((_ROWS, BD), lambda i: (i, 0)),
        out_shape=jax.ShapeDtypeStruct((N, BD), jnp.float32),
    )(p, wt, c)


def _noise_scale(layer_id):
    nkey = jax.random.key(42)
    noise = jax.random.uniform(jax.random.fold_in(nkey, layer_id), (N, BD),
                               dtype=jnp.float32)
    nl = noise / jnp.clip(jnp.linalg.norm(noise, axis=-1, keepdims=True), 1e-12)
    return 1.0 + CL_EPS * nl


def kernel(user_embed, item_embed, W, edge_index, edge_weight):
    x0 = jnp.concatenate([user_embed, item_embed], axis=0)
    wt = W.T

    c0 = _noise_scale(0)
    c1 = _noise_scale(1)
    c2 = _noise_scale(2)

    pad = EPAD - E
    src = jnp.concatenate(
        [edge_index[0], jnp.zeros((pad,), jnp.int32)]).reshape(NW, NCH, CHUNK)
    dst = jnp.concatenate(
        [edge_index[1], jnp.zeros((pad,), jnp.int32)]).reshape(NW, NCH, CHUNK)
    wgt = jnp.concatenate(
        [edge_weight, jnp.zeros((pad,), jnp.float32)]).reshape(NW, NCH, CHUNK)

    b0 = _bin_call(x0, wt, c0)
    p1 = _spmm(x0, src, dst, wgt)
    x1, b1 = _merge_bin_call(p1, wt, c1)
    p2 = _spmm(x1, src, dst, wgt)
    b2 = _merge_bin_last_call(p2, wt, c2)

    emb = jnp.concatenate([b0, b1, b2], axis=1)
    return emb[:NUM_USERS], emb[NUM_USERS:]
